# baseline (speedup 1.0000x reference)
import jax
import jax.numpy as jnp
from jax import lax
from jax.experimental import pallas as pl
from jax.experimental.pallas import tpu as pltpu
from jax.experimental.pallas import tpu_sc as plsc

B = 4096
L = 200
DIM = 32
V = 1000000
NC, NS = 2, 16
NW = NC * NS
LT = L // 8
NBUF = 4
STEP = 4
NS_LOOP = L // STEP
CB = V // 128
VMAIN = CB * 128
CB_IT = ((CB + NW - 1) // NW + 1) // 2 * 2


def _detrans_body(t4_hbm, tail_hbm, out_hbm, s0, s1, o0, o1,
                  gi0, gi1, go0, go1):
    sbuf = (s0, s1)
    obuf = (o0, o1)
    isem = (gi0, gi1)
    osem = (go0, go1)

    wid = lax.axis_index("s") * NC + lax.axis_index("c")

    def cof(i):
        return wid + NW * i

    def fire(i, b):
        @pl.when(cof(i) < CB)
        def _():
            pltpu.async_copy(t4_hbm.at[:, cof(i)], sbuf[b], isem[b])

    def wait_in(i, b):
        @pl.when(cof(i) < CB)
        def _():
            pltpu.make_async_copy(
                t4_hbm.at[:, cof(i)], sbuf[b], isem[b]).wait()

    def store(i, b):
        @pl.when(cof(i) < CB)
        def _():
            pltpu.async_copy(
                obuf[b], out_hbm.at[pl.ds(cof(i) * 128, 128)], osem[b])

    def wait_store(i, b):
        @pl.when(cof(i) < CB)
        def _():
            pltpu.make_async_copy(
                obuf[b], out_hbm.at[pl.ds(cof(i) * 128, 128)],
                osem[b]).wait()

    blidxs = [lax.iota(jnp.int32, 16) + 16 * k for k in range(8)]
    dcols = [jnp.full((16,), d, jnp.int32) for d in range(DIM)]

    def transpose(b):
        for dt in range(4):
            for dr in range(8):
                d = 8 * dt + dr
                vs = [sbuf[b][dt, dr, 16 * k:16 * k + 16]
                      for k in range(8)]
                for k in range(8):
                    plsc.store_scatter(
                        obuf[b], [blidxs[k], dcols[d]], vs[k])

    @pl.when(wid == 0)
    def _():
        pltpu.sync_copy(tail_hbm, out_hbm.at[pl.ds(VMAIN, V - VMAIN)])

    fire(0, 0)
    for i in range(2):
        fire(i + 1, (i + 1) % 2)
        wait_in(i, i % 2)
        transpose(i % 2)
        store(i, i % 2)

    def sbody(s, carry):
        ibase = 2 * s
        for j in range(2):
            i = ibase + j
            fire(i + 1, (j + 1) % 2)
            wait_in(i, j)
            wait_store(i - 2, j)
            transpose(j)
            store(i, j)
        return carry

    lax.fori_loop(1, CB_IT // 2, sbody, 0)

    for j in range(2):
        wait_store(CB_IT - 2 + j, j)


def _gather_body(idx_hbm, table_hbm, out_hbm, idx_v,
                 r0, r1, r2, r3, t0, t1, t2, t3,
                 g0, g1, g2, g3, s0, s1, s2, s3):
    rows = (r0, r1, r2, r3)
    tbuf = (t0, t1, t2, t3)
    gsem = (g0, g1, g2, g3)
    ssem = (s0, s1, s2, s3)

    wid = lax.axis_index("s") * NC + lax.axis_index("c")

    def fire(l, b):
        pltpu.async_copy(
            table_hbm.at[idx_v.at[l // 8, l % 8]], rows[b], gsem[b])

    def wait_gather(l, b):
        del l
        pltpu.make_async_copy(
            table_hbm.at[idx_v.at[0, 0]], rows[b], gsem[b]).wait()

    def store(l, b):
        pltpu.async_copy(tbuf[b], out_hbm.at[l, :, wid], ssem[b])

    def wait_store(l, b):
        pltpu.make_async_copy(
            tbuf[b], out_hbm.at[l, :, wid], ssem[b]).wait()

    bidxs = [lax.iota(jnp.int32, 16) + 16 * k for k in range(8)]
    dcols = [jnp.full((16,), d, jnp.int32) for d in range(DIM)]

    def transpose(b):
        for k in range(8):
            vs = [
                plsc.load_gather(rows[b], [bidxs[k], dcols[d]])
                for d in range(DIM)
            ]
            for d in range(DIM):
                tbuf[b][d // 8, d % 8, 16 * k:16 * k + 16] = vs[d]

    for lt in range(LT):
        pltpu.sync_copy(idx_hbm.at[lt, wid], idx_v.at[lt])

    for b in range(NBUF - 1):
        fire(b, b)

    for j in range(STEP):
        l = j
        fire(l + NBUF - 1, (j + NBUF - 1) % NBUF)
        wait_gather(l, j)
        transpose(j)
        store(l, j)

    def sbody(s, carry):
        lbase = s * STEP
        for j in range(STEP):
            l = lbase + j
            fire(l + NBUF - 1, (j + NBUF - 1) % NBUF)
            wait_gather(l, j)
            wait_store(l - NBUF, j)
            transpose(j)
            store(l, j)
        return carry

    lax.fori_loop(1, NS_LOOP - 1, sbody, 0)

    for j in range(STEP):
        l = L - STEP + j
        if j == 0:
            fire(L - 1, (j + NBUF - 1) % NBUF)
        wait_gather(l, j)
        wait_store(l - NBUF, j)
        transpose(j)
        store(l, j)

    for j in range(NBUF):
        wait_store(L - NBUF + j, j)


_MESH = dict(
    mesh=plsc.VectorSubcoreMesh(core_axis_name="c", subcore_axis_name="s"),
    compiler_params=pltpu.CompilerParams(
        use_tc_tiling_on_sc=False, needs_layout_passes=False),
)


@jax.jit
def kernel(inp, table):
    t4 = (
        table.T[:, :VMAIN]
        .reshape(4, 8, CB, 128)
        .transpose(0, 2, 1, 3)
    )
    tail = table[VMAIN:]
    table_lin = pl.kernel(
        _detrans_body,
        out_type=jax.ShapeDtypeStruct((V, DIM), jnp.float32),
        scratch_types=(
            [pltpu.VMEM((4, 8, 128), jnp.float32) for _ in range(2)]
            + [pltpu.VMEM((128, DIM), jnp.float32) for _ in range(2)]
            + [pltpu.SemaphoreType.DMA for _ in range(4)]
        ),
        **_MESH,
    )(t4, tail)

    idx4 = (
        inp.astype(jnp.int32).T
        .reshape(LT, 8, NW, 128)
        .transpose(0, 2, 1, 3)
    )
    out5 = pl.kernel(
        _gather_body,
        out_type=jax.ShapeDtypeStruct((L, 4, NW, 8, 128), jnp.float32),
        scratch_types=(
            [pltpu.VMEM((LT, 8, 128), jnp.int32)]
            + [pltpu.VMEM((128, DIM), jnp.float32) for _ in range(NBUF)]
            + [pltpu.VMEM((4, 8, 128), jnp.float32) for _ in range(NBUF)]
            + [pltpu.SemaphoreType.DMA for _ in range(2 * NBUF)]
        ),
        **_MESH,
    )(idx4, table_lin)
    return out5.transpose(2, 4, 0, 1, 3).reshape(B, L, DIM)


# speedup vs baseline: 1.2450x; 1.2450x over previous
import jax
import jax.numpy as jnp
from jax import lax
from jax.experimental import pallas as pl
from jax.experimental.pallas import tpu as pltpu
from jax.experimental.pallas import tpu_sc as plsc

B = 4096
L = 200
DIM = 32
NC, NS = 2, 16
NW = NC * NS
LT = L // 8
NBUF = 4
STEP = 4
NS_LOOP = L // STEP
HG = 2


def _body(idx_hbm, table_hbm, out_hbm, idx_v,
          r0, r1, r2, r3, t0, t1, t2, t3,
          g0, g1, g2, g3, s0, s1, s2, s3):
    rows = (r0, r1, r2, r3)
    tbuf = (t0, t1, t2, t3)
    gsem = (g0, g1, g2, g3)
    ssem = (s0, s1, s2, s3)

    wid = lax.axis_index("s") * NC + lax.axis_index("c")
    H = 128 // HG

    def fire(l, b):
        for h in range(HG):
            pltpu.async_copy(
                table_hbm.at[idx_v.at[l // 8, l % 8, pl.ds(h * H, H)]],
                rows[b].at[pl.ds(h * H, H)],
                gsem[b])

    def wait_gather(l, b):
        del l
        for h in range(HG):
            pltpu.make_async_copy(
                table_hbm.at[idx_v.at[0, 0, pl.ds(h * H, H)]],
                rows[b].at[pl.ds(h * H, H)],
                gsem[b]).wait()

    def store(l, b):
        pltpu.async_copy(tbuf[b], out_hbm.at[l, :, wid], ssem[b])

    def wait_store(l, b):
        pltpu.make_async_copy(
            tbuf[b], out_hbm.at[l, :, wid], ssem[b]).wait()

    bidxs = [lax.iota(jnp.int32, 16) + 16 * k for k in range(8)]
    dcols = [jnp.full((16,), d, jnp.int32) for d in range(DIM)]

    def transpose(b):
        for k in range(8):
            vs = [
                plsc.load_gather(rows[b], [bidxs[k], dcols[d]])
                for d in range(DIM)
            ]
            for d in range(DIM):
                tbuf[b][d // 8, d % 8, 16 * k:16 * k + 16] = vs[d]

    for lt in range(LT):
        pltpu.sync_copy(idx_hbm.at[lt, wid], idx_v.at[lt])

    for b in range(NBUF - 1):
        fire(b, b)

    for j in range(STEP):
        l = j
        fire(l + NBUF - 1, (j + NBUF - 1) % NBUF)
        wait_gather(l, j)
        transpose(j)
        store(l, j)

    def sbody(s, carry):
        lbase = s * STEP
        for j in range(STEP):
            l = lbase + j
            fire(l + NBUF - 1, (j + NBUF - 1) % NBUF)
            wait_gather(l, j)
            wait_store(l - NBUF, j)
            transpose(j)
            store(l, j)
        return carry

    lax.fori_loop(1, NS_LOOP - 1, sbody, 0)

    for j in range(STEP):
        l = L - STEP + j
        if j == 0:
            fire(L - 1, (j + NBUF - 1) % NBUF)
        wait_gather(l, j)
        wait_store(l - NBUF, j)
        transpose(j)
        store(l, j)

    for j in range(NBUF):
        wait_store(L - NBUF + j, j)


@jax.jit
def kernel(inp, table):
    idx4 = (
        inp.astype(jnp.int32).T
        .reshape(LT, 8, NW, 128)
        .transpose(0, 2, 1, 3)
    )
    out5 = pl.kernel(
        _body,
        out_type=jax.ShapeDtypeStruct((L, 4, NW, 8, 128), jnp.float32),
        mesh=plsc.VectorSubcoreMesh(core_axis_name="c", subcore_axis_name="s"),
        compiler_params=pltpu.CompilerParams(
            use_tc_tiling_on_sc=False, needs_layout_passes=False),
        scratch_types=[
            pltpu.VMEM((LT, 8, 128), jnp.int32),
        ]
        + [pltpu.VMEM((128, DIM), jnp.float32) for _ in range(NBUF)]
        + [pltpu.VMEM((4, 8, 128), jnp.float32) for _ in range(NBUF)]
        + [pltpu.SemaphoreType.DMA for _ in range(2 * NBUF)],
    )(idx4, table)
    return out5.transpose(2, 4, 0, 1, 3).reshape(B, L, DIM)


# speedup vs baseline: 1.2689x; 1.0192x over previous
import jax
import jax.numpy as jnp
from jax import lax
from jax.experimental import pallas as pl
from jax.experimental.pallas import tpu as pltpu
from jax.experimental.pallas import tpu_sc as plsc

B = 4096
L = 200
DIM = 32
NC, NS = 2, 16
NW = NC * NS
LT = L // 8
NBUF = 8
STEP = 8
NS_LOOP = L // STEP
HG = 2


def _body(idx_hbm, table_hbm, out_hbm, idx_v,
          r0, r1, r2, r3, r4, r5, r6, r7,
          t0, t1, t2, t3, t4, t5, t6, t7,
          g0, g1, g2, g3, g4, g5, g6, g7,
          s0, s1, s2, s3, s4, s5, s6, s7):
    rows = (r0, r1, r2, r3, r4, r5, r6, r7)
    tbuf = (t0, t1, t2, t3, t4, t5, t6, t7)
    gsem = (g0, g1, g2, g3, g4, g5, g6, g7)
    ssem = (s0, s1, s2, s3, s4, s5, s6, s7)

    wid = lax.axis_index("s") * NC + lax.axis_index("c")
    H = 128 // HG

    def fire(l, b):
        for h in range(HG):
            pltpu.async_copy(
                table_hbm.at[idx_v.at[l // 8, l % 8, pl.ds(h * H, H)]],
                rows[b].at[pl.ds(h * H, H)],
                gsem[b])

    def wait_gather(l, b):
        del l
        for h in range(HG):
            pltpu.make_async_copy(
                table_hbm.at[idx_v.at[0, 0, pl.ds(h * H, H)]],
                rows[b].at[pl.ds(h * H, H)],
                gsem[b]).wait()

    def store(l, b):
        pltpu.async_copy(tbuf[b], out_hbm.at[l, :, wid], ssem[b])

    def wait_store(l, b):
        pltpu.make_async_copy(
            tbuf[b], out_hbm.at[l, :, wid], ssem[b]).wait()

    dcols = [jnp.full((16,), d, jnp.int32) for d in range(DIM)]

    def transpose(b):
        def kbody(k, carry):
            bidx = lax.iota(jnp.int32, 16) + 16 * k
            vs = [
                plsc.load_gather(rows[b], [bidx, dcols[d]])
                for d in range(DIM)
            ]
            for d in range(DIM):
                tbuf[b][d // 8, d % 8, pl.ds(16 * k, 16)] = vs[d]
            return carry

        lax.fori_loop(0, 8, kbody, 0)

    for lt in range(LT):
        pltpu.sync_copy(idx_hbm.at[lt, wid], idx_v.at[lt])

    for b in range(NBUF - 1):
        fire(b, b)

    for j in range(STEP):
        l = j
        fire(l + NBUF - 1, (j + NBUF - 1) % NBUF)
        wait_gather(l, j)
        transpose(j)
        store(l, j)

    def sbody(s, carry):
        lbase = s * STEP
        for j in range(STEP):
            l = lbase + j
            fire(l + NBUF - 1, (j + NBUF - 1) % NBUF)
            wait_gather(l, j)
            wait_store(l - NBUF, j)
            transpose(j)
            store(l, j)
        return carry

    lax.fori_loop(1, NS_LOOP - 1, sbody, 0)

    for j in range(STEP):
        l = L - STEP + j
        if j == 0:
            fire(L - 1, (j + NBUF - 1) % NBUF)
        wait_gather(l, j)
        wait_store(l - NBUF, j)
        transpose(j)
        store(l, j)

    for j in range(NBUF):
        wait_store(L - NBUF + j, j)


@jax.jit
def kernel(inp, table):
    idx4 = (
        inp.astype(jnp.int32).T
        .reshape(LT, 8, NW, 128)
        .transpose(0, 2, 1, 3)
    )
    out5 = pl.kernel(
        _body,
        out_type=jax.ShapeDtypeStruct((L, 4, NW, 8, 128), jnp.float32),
        mesh=plsc.VectorSubcoreMesh(core_axis_name="c", subcore_axis_name="s"),
        compiler_params=pltpu.CompilerParams(
            use_tc_tiling_on_sc=False, needs_layout_passes=False),
        scratch_types=[
            pltpu.VMEM((LT, 8, 128), jnp.int32),
        ]
        + [pltpu.VMEM((128, DIM), jnp.float32) for _ in range(NBUF)]
        + [pltpu.VMEM((4, 8, 128), jnp.float32) for _ in range(NBUF)]
        + [pltpu.SemaphoreType.DMA for _ in range(2 * NBUF)],
    )(idx4, table)
    return out5.transpose(2, 4, 0, 1, 3).reshape(B, L, DIM)
